# Initial kernel scaffold; baseline (speedup 1.0000x reference)
import jax
import jax.numpy as jnp
from jax.experimental import pallas as pl


def kernel(x, router_w, up_proj, down_proj):
    raise NotImplementedError("write your pallas kernel here")


# speedup vs baseline: 4.0103x; 4.0103x over previous
import functools

import jax
import jax.numpy as jnp
from jax.experimental import pallas as pl
from jax.experimental.pallas import tpu as pltpu

_E = 64
_K = 2
_H = 1024
_F = 512


def _route_body(x_ref, w_ref, i1_ref, i2_ref, w1_ref, w2_ref):
    x = x_ref[...]
    w = w_ref[...]
    logits = jax.lax.dot_general(
        x, w, (((1,), (1,)), ((), ())),
        preferred_element_type=jnp.float32)
    cols = jax.lax.broadcasted_iota(jnp.int32, logits.shape, 1)
    m1 = jnp.max(logits, axis=1, keepdims=True)
    i1 = jnp.min(jnp.where(logits == m1, cols, _E), axis=1, keepdims=True)
    masked = jnp.where(cols == i1, -jnp.inf, logits)
    m2 = jnp.max(masked, axis=1, keepdims=True)
    i2 = jnp.min(jnp.where(masked == m2, cols, _E), axis=1, keepdims=True)
    w1 = 1.0 / (1.0 + jnp.exp(m2 - m1))
    i1_ref[...] = i1
    i2_ref[...] = i2
    w1_ref[...] = w1
    w2_ref[...] = 1.0 - w1


def _moe_body(i1_ref, i2_ref, w1_ref, w2_ref, x_ref, up_ref, dn_ref, o_ref):
    e = pl.program_id(0)

    @pl.when(e == 0)
    def _init():
        o_ref[...] = jnp.zeros_like(o_ref)

    xb = x_ref[...].astype(jnp.bfloat16)
    up = up_ref[0].astype(jnp.bfloat16)
    h = jnp.dot(xb, up, preferred_element_type=jnp.float32)
    h1 = h[:, :_F]
    h2 = h[:, _F:]
    act = (h1 * jax.nn.sigmoid(h1) * h2).astype(jnp.bfloat16)
    dn = dn_ref[0].astype(jnp.bfloat16)
    y = jnp.dot(act, dn, preferred_element_type=jnp.float32)
    col = (jnp.where(i1_ref[...] == e, w1_ref[...], 0.0)
           + jnp.where(i2_ref[...] == e, w2_ref[...], 0.0))
    o_ref[...] += col * y


def kernel(x, router_w, up_proj, down_proj):
    b, s, h = x.shape
    xf = x.reshape(-1, h)
    t = xf.shape[0]

    i1, i2, w1, w2 = pl.pallas_call(
        _route_body,
        out_shape=(
            jax.ShapeDtypeStruct((t, 1), jnp.int32),
            jax.ShapeDtypeStruct((t, 1), jnp.int32),
            jax.ShapeDtypeStruct((t, 1), jnp.float32),
            jax.ShapeDtypeStruct((t, 1), jnp.float32),
        ),
    )(xf, router_w)

    out = pl.pallas_call(
        _moe_body,
        grid=(_E,),
        in_specs=[
            pl.BlockSpec((t, 1), lambda e: (0, 0)),
            pl.BlockSpec((t, 1), lambda e: (0, 0)),
            pl.BlockSpec((t, 1), lambda e: (0, 0)),
            pl.BlockSpec((t, 1), lambda e: (0, 0)),
            pl.BlockSpec((t, h), lambda e: (0, 0)),
            pl.BlockSpec((1, _H, 2 * _F), lambda e: (e, 0, 0)),
            pl.BlockSpec((1, _F, _H), lambda e: (e, 0, 0)),
        ],
        out_specs=pl.BlockSpec((t, h), lambda e: (0, 0)),
        out_shape=jax.ShapeDtypeStruct((t, h), jnp.float32),
    )(i1, i2, w1, w2, xf, up_proj, down_proj)

    return out.reshape(b, s, h)
